# Initial kernel scaffold; baseline (speedup 1.0000x reference)
import jax
import jax.numpy as jnp
from jax.experimental import pallas as pl


def kernel(x, pos, edge_index, edge_attr, batch, g, mu, sigma, root_w, root_b, lin1_w, lin1_b, lin2_w, lin2_b):
    raise NotImplementedError("write your pallas kernel here")


# speedup vs baseline: 1.7686x; 1.7686x over previous
import functools

import jax
import jax.numpy as jnp
from jax import lax
from jax.experimental import pallas as pl
from jax.experimental.pallas import tpu as pltpu
from jax.experimental.pallas import tpu_sc as plsc

N = 10000
E = 320000
D_IN = 130
K = 5
H = 64
G = 16
C = 10

W = 80
CH = 128
NCH = E // CH
NW = 32
ITERS = (NCH + NW - 1) // NW
ROWS_PER_TILE = N // 16


def _mm_body(h_ref, g_ref, o_ref):
    o_ref[...] = jnp.dot(h_ref[...], g_ref[...],
                         preferred_element_type=jnp.float32)


def _compute_hg(h, g):
    return pl.pallas_call(
        _mm_body,
        grid=(10,),
        in_specs=[
            pl.BlockSpec((N // 10, D_IN), lambda i: (i, 0)),
            pl.BlockSpec((D_IN, K * H), lambda i: (0, 0)),
        ],
        out_specs=pl.BlockSpec((N // 10, K * H), lambda i: (i, 0)),
        out_shape=jax.ShapeDtypeStruct((N, K * H), jnp.float32),
    )(h, g)


def _edge_body(hg_hbm, src_hbm, dst_hbm, ea_hbm, prm_hbm, out_hbm,
               src_v, dst_v, ea_v, prm_v, rows_v, msg_v, zbuf, num_sh, gsem):
    cid = lax.axis_index("c")
    sid = lax.axis_index("s")
    wid = cid * 16 + sid

    iota16 = jnp.arange(16, dtype=jnp.int32)

    zvec = jnp.zeros((16,), jnp.float32)
    for r in range(25):
        for c in range(W // 16):
            zbuf[r, pl.ds(c * 16, 16)] = zvec

    def _zero_step(j, carry):
        pltpu.sync_copy(zbuf, num_sh.at[pl.ds(sid * ROWS_PER_TILE + j * 25, 25), :])
        return carry

    lax.fori_loop(0, ROWS_PER_TILE // 25, _zero_step, 0)

    pltpu.sync_copy(prm_hbm, prm_v)
    mu0 = [prm_v[k] for k in range(K)]
    mu1 = [prm_v[K + k] for k in range(K)]
    c0 = [prm_v[2 * K + k] for k in range(K)]
    c1 = [prm_v[3 * K + k] for k in range(K)]

    cnt_vec = jnp.where(iota16 == 0, 1.0, 0.0).astype(jnp.float32)
    for e in range(CH):
        msg_v[e, pl.ds(H, 16)] = cnt_vec

    plsc.subcore_barrier()

    def _chunk(i, carry):
        ci = i * NW + wid

        @pl.when(ci < NCH)
        def _():
            base = ci * CH
            pltpu.sync_copy(src_hbm.at[pl.ds(base, CH)], src_v)
            pltpu.sync_copy(dst_hbm.at[pl.ds(base, CH)], dst_v)
            pltpu.sync_copy(ea_hbm.at[pl.ds(base * 2, CH * 2)], ea_v)
            pltpu.async_copy(hg_hbm.at[src_v], rows_v, gsem).wait()

            def _group(eg, c2):
                lane = iota16 + eg * 16
                a0 = plsc.load_gather(ea_v, [lane * 2])
                a1 = plsc.load_gather(ea_v, [lane * 2 + 1])
                ws = []
                for k in range(K):
                    d0 = a0 - mu0[k]
                    d1 = a1 - mu1[k]
                    ws.append(jnp.exp(d0 * d0 * c0[k] + d1 * d1 * c1[k]))
                for hc in range(H):
                    acc = ws[0] * plsc.load_gather(
                        rows_v, [lane, jnp.full((16,), hc, jnp.int32)])
                    for k in range(1, K):
                        acc = acc + ws[k] * plsc.load_gather(
                            rows_v, [lane, jnp.full((16,), k * H + hc, jnp.int32)])
                    plsc.store_scatter(
                        msg_v, [lane, jnp.full((16,), hc, jnp.int32)], acc)
                return c2

            lax.fori_loop(0, CH // 16, _group, 0)
            pltpu.sync_copy(msg_v, num_sh.at[dst_v], add=True)

        return carry

    lax.fori_loop(0, ITERS, _chunk, 0)

    plsc.subcore_barrier()

    @pl.when(sid == 0)
    def _():
        pltpu.sync_copy(num_sh, out_hbm.at[cid])


def _edge_aggregate(hg, src, dst, ea_flat, prm):
    mesh = plsc.VectorSubcoreMesh(core_axis_name="c", subcore_axis_name="s")
    k = pl.kernel(
        _edge_body,
        out_type=jax.ShapeDtypeStruct((2, N, W), jnp.float32),
        mesh=mesh,
        compiler_params=pltpu.CompilerParams(needs_layout_passes=False,
                                             use_tc_tiling_on_sc=False),
        scratch_types=[
            pltpu.VMEM((CH,), jnp.int32),
            pltpu.VMEM((CH,), jnp.int32),
            pltpu.VMEM((CH * 2,), jnp.float32),
            pltpu.VMEM((4 * K, 16), jnp.float32),
            pltpu.VMEM((CH, K * H), jnp.float32),
            pltpu.VMEM((CH, W), jnp.float32),
            pltpu.VMEM((25, W), jnp.float32),
            pltpu.VMEM_SHARED((N, W), jnp.float32),
            pltpu.SemaphoreType.DMA,
        ],
    )
    return k(hg, src, dst, ea_flat, prm)


def _head_body(parts_ref, h_ref, batch_ref, root_w_ref, root_b_ref,
               lin1_w_ref, lin1_b_ref, lin2_w_ref, lin2_b_ref, o_ref,
               acc_msg, acc_h, acc_n):
    i = pl.program_id(0)

    @pl.when(i == 0)
    def _():
        acc_msg[...] = jnp.zeros_like(acc_msg)
        acc_h[...] = jnp.zeros_like(acc_h)
        acc_n[...] = jnp.zeros_like(acc_n)

    num = parts_ref[0, :, :H] + parts_ref[1, :, :H]
    cnt = parts_ref[0, :, H] + parts_ref[1, :, H]
    out_node = num / jnp.maximum(cnt, 1.0)[:, None]

    b = batch_ref[0, 0, :]
    oh = (b[None, :] == lax.broadcasted_iota(jnp.int32, (G, b.shape[0]), 0))
    oh = oh.astype(jnp.float32)
    acc_msg[...] += jnp.dot(oh, out_node, preferred_element_type=jnp.float32)
    acc_h[...] += jnp.dot(oh, h_ref[...], preferred_element_type=jnp.float32)
    acc_n[0, :] += jnp.sum(oh, axis=1)

    @pl.when(i == pl.num_programs(0) - 1)
    def _():
        nb = acc_n[0, :]
        pooled = (acc_msg[...]
                  + jnp.dot(acc_h[...], root_w_ref[...],
                            preferred_element_type=jnp.float32)
                  + nb[:, None] * root_b_ref[...])
        pooled = pooled / jnp.maximum(nb, 1.0)[:, None]
        z = jnp.maximum(
            jnp.dot(pooled, lin1_w_ref[...],
                    preferred_element_type=jnp.float32) + lin1_b_ref[...],
            0.0)
        logits = jnp.dot(z, lin2_w_ref[...],
                         preferred_element_type=jnp.float32) + lin2_b_ref[...]
        m = jnp.max(logits, axis=1, keepdims=True)
        s = jnp.log(jnp.sum(jnp.exp(logits - m), axis=1, keepdims=True))
        o_ref[...] = logits - m - s


def _head(parts, h, batch3, root_w, root_b, lin1_w, lin1_b, lin2_w, lin2_b):
    nb = 10
    rows = N // nb
    return pl.pallas_call(
        _head_body,
        grid=(nb,),
        in_specs=[
            pl.BlockSpec((2, rows, W), lambda i: (0, i, 0)),
            pl.BlockSpec((rows, D_IN), lambda i: (i, 0)),
            pl.BlockSpec((1, 1, rows), lambda i: (i, 0, 0)),
            pl.BlockSpec((D_IN, H), lambda i: (0, 0)),
            pl.BlockSpec((1, H), lambda i: (0, 0)),
            pl.BlockSpec((H, H), lambda i: (0, 0)),
            pl.BlockSpec((1, H), lambda i: (0, 0)),
            pl.BlockSpec((H, C), lambda i: (0, 0)),
            pl.BlockSpec((1, C), lambda i: (0, 0)),
        ],
        out_specs=pl.BlockSpec((G, C), lambda i: (0, 0)),
        out_shape=jax.ShapeDtypeStruct((G, C), jnp.float32),
        scratch_shapes=[
            pltpu.VMEM((G, H), jnp.float32),
            pltpu.VMEM((G, D_IN), jnp.float32),
            pltpu.VMEM((1, G), jnp.float32),
        ],
    )(parts, h, batch3, root_w, root_b, lin1_w, lin1_b, lin2_w, lin2_b)


def kernel(x, pos, edge_index, edge_attr, batch, g, mu, sigma, root_w, root_b,
           lin1_w, lin1_b, lin2_w, lin2_b):
    h = jnp.concatenate([pos, x], axis=1)
    hg = _compute_hg(h, g)

    src = edge_index[0].astype(jnp.int32)
    dst = edge_index[1].astype(jnp.int32)
    ea_flat = edge_attr.reshape(-1)

    inv0 = -0.5 / (1e-15 + sigma[:, 0] ** 2)
    inv1 = -0.5 / (1e-15 + sigma[:, 1] ** 2)
    prm = jnp.concatenate([mu[:, 0], mu[:, 1], inv0, inv1])
    prm = jnp.broadcast_to(prm[:, None], (4 * K, 16)).astype(jnp.float32)

    parts = _edge_aggregate(hg, src, dst, ea_flat, prm)

    batch3 = batch.astype(jnp.int32).reshape(10, 1, N // 10)
    return _head(parts, h, batch3, root_w,
                 root_b.reshape(1, H), lin1_w, lin1_b.reshape(1, H),
                 lin2_w, lin2_b.reshape(1, C))


# speedup vs baseline: 4.3633x; 2.4671x over previous
import functools

import jax
import jax.numpy as jnp
from jax import lax
from jax.experimental import pallas as pl
from jax.experimental.pallas import tpu as pltpu
from jax.experimental.pallas import tpu_sc as plsc

N = 10000
E = 320000
D_IN = 130
K = 5
H = 64
G = 16
C = 10

W = 80
CH = 128
NCH = E // CH
NW = 32
ITERS = (NCH + NW - 1) // NW
ROWS_PER_TILE = N // 16


def _mm_body(h_ref, g_ref, o_ref):
    o_ref[...] = jnp.dot(h_ref[...], g_ref[...],
                         preferred_element_type=jnp.float32)


def _compute_hg(h, g):
    return pl.pallas_call(
        _mm_body,
        grid=(10,),
        in_specs=[
            pl.BlockSpec((N // 10, D_IN), lambda i: (i, 0)),
            pl.BlockSpec((D_IN, K * H), lambda i: (0, 0)),
        ],
        out_specs=pl.BlockSpec((N // 10, K * H), lambda i: (i, 0)),
        out_shape=jax.ShapeDtypeStruct((N, K * H), jnp.float32),
    )(h, g)


def _edge_body(hg_hbm, src_hbm, dst_hbm, ea_hbm, prm_hbm, out_hbm,
               src_v, dst_v, ea_v, prm_v, rows_v, msg_v, zbuf, num_sh, gsem):
    cid = lax.axis_index("c")
    sid = lax.axis_index("s")
    wid = cid * 16 + sid

    iota16 = jnp.arange(16, dtype=jnp.int32)

    zvec = jnp.zeros((16,), jnp.float32)
    for r in range(25):
        for c in range(W // 16):
            zbuf[r, pl.ds(c * 16, 16)] = zvec

    def _zero_step(j, carry):
        pltpu.sync_copy(zbuf, num_sh.at[pl.ds(sid * ROWS_PER_TILE + j * 25, 25), :])
        return carry

    lax.fori_loop(0, ROWS_PER_TILE // 25, _zero_step, 0)

    pltpu.sync_copy(prm_hbm, prm_v)
    mu0 = [prm_v[k] for k in range(K)]
    mu1 = [prm_v[K + k] for k in range(K)]
    c0 = [prm_v[2 * K + k] for k in range(K)]
    c1 = [prm_v[3 * K + k] for k in range(K)]

    cnt_vec = jnp.where(iota16 == 0, 1.0, 0.0).astype(jnp.float32)
    for e in range(CH):
        msg_v[e, pl.ds(H, 16)] = cnt_vec

    plsc.subcore_barrier()

    def _chunk(i, carry):
        ci = i * NW + wid

        @pl.when(ci < NCH)
        def _():
            base = ci * CH
            pltpu.sync_copy(src_hbm.at[pl.ds(base, CH)], src_v)
            pltpu.sync_copy(dst_hbm.at[pl.ds(base, CH)], dst_v)
            pltpu.sync_copy(ea_hbm.at[pl.ds(base * 2, CH * 2)], ea_v)
            pltpu.async_copy(hg_hbm.at[src_v], rows_v, gsem).wait()

            def _group(eg, c2):
                lane = iota16 + eg * 16
                a0 = plsc.load_gather(ea_v, [lane * 2])
                a1 = plsc.load_gather(ea_v, [lane * 2 + 1])
                ws = []
                for k in range(K):
                    d0 = a0 - mu0[k]
                    d1 = a1 - mu1[k]
                    ws.append(jnp.exp(d0 * d0 * c0[k] + d1 * d1 * c1[k]))
                for hc in range(H):
                    hb, j = (hc >> 4) * 16, hc & 15
                    rot = hb + ((iota16 + j) & 15)
                    acc = ws[0] * plsc.load_gather(rows_v, [lane, rot])
                    for k in range(1, K):
                        acc = acc + ws[k] * plsc.load_gather(
                            rows_v, [lane, k * H + rot])
                    plsc.store_scatter(msg_v, [lane, rot], acc)
                return c2

            lax.fori_loop(0, CH // 16, _group, 0)
            pltpu.sync_copy(msg_v, num_sh.at[dst_v], add=True)

        return carry

    lax.fori_loop(0, ITERS, _chunk, 0)

    plsc.subcore_barrier()

    @pl.when(sid == 0)
    def _():
        pltpu.sync_copy(num_sh, out_hbm.at[cid])


def _edge_aggregate(hg, src, dst, ea_flat, prm):
    mesh = plsc.VectorSubcoreMesh(core_axis_name="c", subcore_axis_name="s")
    k = pl.kernel(
        _edge_body,
        out_type=jax.ShapeDtypeStruct((2, N, W), jnp.float32),
        mesh=mesh,
        compiler_params=pltpu.CompilerParams(needs_layout_passes=False,
                                             use_tc_tiling_on_sc=False),
        scratch_types=[
            pltpu.VMEM((CH,), jnp.int32),
            pltpu.VMEM((CH,), jnp.int32),
            pltpu.VMEM((CH * 2,), jnp.float32),
            pltpu.VMEM((4 * K, 16), jnp.float32),
            pltpu.VMEM((CH, K * H), jnp.float32),
            pltpu.VMEM((CH, W), jnp.float32),
            pltpu.VMEM((25, W), jnp.float32),
            pltpu.VMEM_SHARED((N, W), jnp.float32),
            pltpu.SemaphoreType.DMA,
        ],
    )
    return k(hg, src, dst, ea_flat, prm)


def _head_body(parts_ref, h_ref, batch_ref, root_w_ref, root_b_ref,
               lin1_w_ref, lin1_b_ref, lin2_w_ref, lin2_b_ref, o_ref,
               acc_msg, acc_h, acc_n):
    i = pl.program_id(0)

    @pl.when(i == 0)
    def _():
        acc_msg[...] = jnp.zeros_like(acc_msg)
        acc_h[...] = jnp.zeros_like(acc_h)
        acc_n[...] = jnp.zeros_like(acc_n)

    num = parts_ref[0, :, :H] + parts_ref[1, :, :H]
    cnt = parts_ref[0, :, H] + parts_ref[1, :, H]
    out_node = num / jnp.maximum(cnt, 1.0)[:, None]

    b = batch_ref[0, 0, :]
    oh = (b[None, :] == lax.broadcasted_iota(jnp.int32, (G, b.shape[0]), 0))
    oh = oh.astype(jnp.float32)
    acc_msg[...] += jnp.dot(oh, out_node, preferred_element_type=jnp.float32)
    acc_h[...] += jnp.dot(oh, h_ref[...], preferred_element_type=jnp.float32)
    acc_n[0, :] += jnp.sum(oh, axis=1)

    @pl.when(i == pl.num_programs(0) - 1)
    def _():
        nb = acc_n[0, :]
        pooled = (acc_msg[...]
                  + jnp.dot(acc_h[...], root_w_ref[...],
                            preferred_element_type=jnp.float32)
                  + nb[:, None] * root_b_ref[...])
        pooled = pooled / jnp.maximum(nb, 1.0)[:, None]
        z = jnp.maximum(
            jnp.dot(pooled, lin1_w_ref[...],
                    preferred_element_type=jnp.float32) + lin1_b_ref[...],
            0.0)
        logits = jnp.dot(z, lin2_w_ref[...],
                         preferred_element_type=jnp.float32) + lin2_b_ref[...]
        m = jnp.max(logits, axis=1, keepdims=True)
        s = jnp.log(jnp.sum(jnp.exp(logits - m), axis=1, keepdims=True))
        o_ref[...] = logits - m - s


def _head(parts, h, batch3, root_w, root_b, lin1_w, lin1_b, lin2_w, lin2_b):
    nb = 10
    rows = N // nb
    return pl.pallas_call(
        _head_body,
        grid=(nb,),
        in_specs=[
            pl.BlockSpec((2, rows, W), lambda i: (0, i, 0)),
            pl.BlockSpec((rows, D_IN), lambda i: (i, 0)),
            pl.BlockSpec((1, 1, rows), lambda i: (i, 0, 0)),
            pl.BlockSpec((D_IN, H), lambda i: (0, 0)),
            pl.BlockSpec((1, H), lambda i: (0, 0)),
            pl.BlockSpec((H, H), lambda i: (0, 0)),
            pl.BlockSpec((1, H), lambda i: (0, 0)),
            pl.BlockSpec((H, C), lambda i: (0, 0)),
            pl.BlockSpec((1, C), lambda i: (0, 0)),
        ],
        out_specs=pl.BlockSpec((G, C), lambda i: (0, 0)),
        out_shape=jax.ShapeDtypeStruct((G, C), jnp.float32),
        scratch_shapes=[
            pltpu.VMEM((G, H), jnp.float32),
            pltpu.VMEM((G, D_IN), jnp.float32),
            pltpu.VMEM((1, G), jnp.float32),
        ],
    )(parts, h, batch3, root_w, root_b, lin1_w, lin1_b, lin2_w, lin2_b)


def kernel(x, pos, edge_index, edge_attr, batch, g, mu, sigma, root_w, root_b,
           lin1_w, lin1_b, lin2_w, lin2_b):
    h = jnp.concatenate([pos, x], axis=1)
    hg = _compute_hg(h, g)

    src = edge_index[0].astype(jnp.int32)
    dst = edge_index[1].astype(jnp.int32)
    ea_flat = edge_attr.reshape(-1)

    inv0 = -0.5 / (1e-15 + sigma[:, 0] ** 2)
    inv1 = -0.5 / (1e-15 + sigma[:, 1] ** 2)
    prm = jnp.concatenate([mu[:, 0], mu[:, 1], inv0, inv1])
    prm = jnp.broadcast_to(prm[:, None], (4 * K, 16)).astype(jnp.float32)

    parts = _edge_aggregate(hg, src, dst, ea_flat, prm)

    batch3 = batch.astype(jnp.int32).reshape(10, 1, N // 10)
    return _head(parts, h, batch3, root_w,
                 root_b.reshape(1, H), lin1_w, lin1_b.reshape(1, H),
                 lin2_w, lin2_b.reshape(1, C))


# speedup vs baseline: 6.9585x; 1.5948x over previous
import functools

import jax
import jax.numpy as jnp
from jax import lax
from jax.experimental import pallas as pl
from jax.experimental.pallas import tpu as pltpu
from jax.experimental.pallas import tpu_sc as plsc

N = 10000
E = 320000
D_IN = 130
K = 5
H = 64
G = 16
C = 10

W = 80
CH = 80
NW = 32
EPW = E // NW
CPW = EPW // CH
ROWS_PER_TILE = N // 16


def _mm_body(h_ref, g_ref, o_ref):
    o_ref[...] = jnp.dot(h_ref[...], g_ref[...],
                         preferred_element_type=jnp.float32
                         ).astype(jnp.bfloat16)


def _compute_hg(h, g):
    return pl.pallas_call(
        _mm_body,
        grid=(10,),
        in_specs=[
            pl.BlockSpec((N // 10, D_IN), lambda i: (i, 0)),
            pl.BlockSpec((D_IN, K * H), lambda i: (0, 0)),
        ],
        out_specs=pl.BlockSpec((N // 10, K * H), lambda i: (i, 0)),
        out_shape=jax.ShapeDtypeStruct((N, K * H), jnp.bfloat16),
    )(h, g)


def _edge_body(hg_hbm, src_hbm, dst_hbm, ea_hbm, prm_hbm, out_hbm,
               sidx_v, didx_v, ea_v, prm_v, rows0, rows1, msg0, msg1,
               didx_c0, didx_c1, zbuf, num_sh, gsem0, gsem1, ssem0, ssem1):
    cid = lax.axis_index("c")
    sid = lax.axis_index("s")
    wid = cid * 16 + sid
    ebase = wid * EPW

    iota16 = jnp.arange(16, dtype=jnp.int32)

    zvec = jnp.zeros((16,), jnp.float32)
    for r in range(25):
        for c in range(W // 16):
            zbuf[r, pl.ds(c * 16, 16)] = zvec

    def _zero_step(j, carry):
        pltpu.sync_copy(zbuf, num_sh.at[pl.ds(sid * ROWS_PER_TILE + j * 25, 25), :])
        return carry

    lax.fori_loop(0, ROWS_PER_TILE // 25, _zero_step, 0)

    pltpu.sync_copy(prm_hbm, prm_v)
    mu0 = [prm_v[k] for k in range(K)]
    mu1 = [prm_v[K + k] for k in range(K)]
    c0 = [prm_v[2 * K + k] for k in range(K)]
    c1 = [prm_v[3 * K + k] for k in range(K)]

    cnt_vec = jnp.where(iota16 == 0, 1.0, 0.0).astype(jnp.float32)
    for msg in (msg0, msg1):
        for e in range(CH):
            msg[e, pl.ds(H, 16)] = cnt_vec

    pltpu.sync_copy(src_hbm.at[pl.ds(ebase, EPW)], sidx_v)
    pltpu.sync_copy(dst_hbm.at[pl.ds(ebase, EPW)], didx_v)
    pltpu.sync_copy(ea_hbm.at[pl.ds(ebase * 2, EPW * 2)], ea_v)

    plsc.subcore_barrier()

    def _gather(c, rows, sem):
        return pltpu.async_copy(
            hg_hbm.at[sidx_v.at[pl.ds(c * CH, CH)]], rows, sem)

    def _wait_gather(c, rows, sem):
        pltpu.make_async_copy(
            hg_hbm.at[sidx_v.at[pl.ds(c * CH, CH)]], rows, sem).wait()

    def _fill_didx(c, didx_c):
        for j in range(CH // 16):
            didx_c[pl.ds(j * 16, 16)] = didx_v[pl.ds(c * CH + j * 16, 16)]

    def _wait_scatter(msg, didx_c, sem):
        pltpu.make_async_copy(msg, num_sh.at[didx_c], sem).wait()

    def _compute(c, rows, msg):
        def _group(eg, c2):
            lane = iota16 + eg * 16
            eidx = (c * CH + eg * 16 + iota16) * 2
            a0 = plsc.load_gather(ea_v, [eidx])
            a1 = plsc.load_gather(ea_v, [eidx + 1])
            ws = []
            for k in range(K):
                d0 = a0 - mu0[k]
                d1 = a1 - mu1[k]
                ws.append(jnp.exp(d0 * d0 * c0[k] + d1 * d1 * c1[k]))
            def _diag(j, c3):
                rotj = (iota16 + j) & 15
                for mb in range(2):
                    rot = mb * 16 + rotj
                    acc_e = None
                    acc_o = None
                    for k in range(K):
                        v = plsc.load_gather(rows, [lane, k * (H // 2) + rot])
                        pair = plsc.bitcast(v, jnp.bfloat16)
                        lo, hi = plsc.unpack(
                            pair, format=plsc.PackFormat.INTERLEAVED)
                        if acc_e is None:
                            acc_e = ws[k] * lo
                            acc_o = ws[k] * hi
                        else:
                            acc_e = acc_e + ws[k] * lo
                            acc_o = acc_o + ws[k] * hi
                    plsc.store_scatter(msg, [lane, 2 * rot], acc_e)
                    plsc.store_scatter(msg, [lane, 2 * rot + 1], acc_o)
                return c3

            lax.fori_loop(0, 16, _diag, 0)
            return c2

        lax.fori_loop(0, CH // 16, _group, 0)

    _gather(0, rows0, gsem0)

    def _pair(p, carry):
        ca = 2 * p
        cb = 2 * p + 1
        _wait_gather(ca, rows0, gsem0)
        _gather(cb, rows1, gsem1)

        @pl.when(p > 0)
        def _():
            _wait_scatter(msg0, didx_c0, ssem0)

        _fill_didx(ca, didx_c0)
        _compute(ca, rows0, msg0)
        pltpu.async_copy(msg0, num_sh.at[didx_c0], ssem0, add=True)

        _wait_gather(cb, rows1, gsem1)
        _gather(ca + 2, rows0, gsem0)

        @pl.when(p > 0)
        def _():
            _wait_scatter(msg1, didx_c1, ssem1)

        _fill_didx(cb, didx_c1)
        _compute(cb, rows1, msg1)
        pltpu.async_copy(msg1, num_sh.at[didx_c1], ssem1, add=True)
        return carry

    lax.fori_loop(0, (CPW - 1) // 2, _pair, 0)

    c_last = CPW - 1
    _wait_gather(c_last, rows0, gsem0)
    _wait_scatter(msg0, didx_c0, ssem0)
    _fill_didx(c_last, didx_c0)
    _compute(c_last, rows0, msg0)
    pltpu.async_copy(msg0, num_sh.at[didx_c0], ssem0, add=True)
    _wait_scatter(msg0, didx_c0, ssem0)
    _wait_scatter(msg1, didx_c1, ssem1)

    plsc.subcore_barrier()

    @pl.when(sid == 0)
    def _():
        pltpu.sync_copy(num_sh, out_hbm.at[cid])


def _edge_aggregate(hgp, src, dst, ea_flat, prm):
    mesh = plsc.VectorSubcoreMesh(core_axis_name="c", subcore_axis_name="s")
    k = pl.kernel(
        _edge_body,
        out_type=jax.ShapeDtypeStruct((2, N, W), jnp.float32),
        mesh=mesh,
        compiler_params=pltpu.CompilerParams(needs_layout_passes=False,
                                             use_tc_tiling_on_sc=False,
                                             internal_scratch_in_bytes=64 * 1024),
        scratch_types=[
            pltpu.VMEM((EPW,), jnp.int32),
            pltpu.VMEM((EPW,), jnp.int32),
            pltpu.VMEM((EPW * 2,), jnp.float32),
            pltpu.VMEM((4 * K, 16), jnp.float32),
            pltpu.VMEM((CH, K * H // 2), jnp.int32),
            pltpu.VMEM((CH, K * H // 2), jnp.int32),
            pltpu.VMEM((CH, W), jnp.float32),
            pltpu.VMEM((CH, W), jnp.float32),
            pltpu.VMEM((CH,), jnp.int32),
            pltpu.VMEM((CH,), jnp.int32),
            pltpu.VMEM((25, W), jnp.float32),
            pltpu.VMEM_SHARED((N, W), jnp.float32),
            pltpu.SemaphoreType.DMA,
            pltpu.SemaphoreType.DMA,
            pltpu.SemaphoreType.DMA,
            pltpu.SemaphoreType.DMA,
        ],
    )
    return k(hgp, src, dst, ea_flat, prm)


def _head_body(parts_ref, h_ref, batch_ref, root_w_ref, root_b_ref,
               lin1_w_ref, lin1_b_ref, lin2_w_ref, lin2_b_ref, o_ref,
               acc_msg, acc_h, acc_n):
    i = pl.program_id(0)

    @pl.when(i == 0)
    def _():
        acc_msg[...] = jnp.zeros_like(acc_msg)
        acc_h[...] = jnp.zeros_like(acc_h)
        acc_n[...] = jnp.zeros_like(acc_n)

    num = parts_ref[0, :, :H] + parts_ref[1, :, :H]
    cnt = parts_ref[0, :, H] + parts_ref[1, :, H]
    out_node = num / jnp.maximum(cnt, 1.0)[:, None]

    b = batch_ref[0, 0, :]
    oh = (b[None, :] == lax.broadcasted_iota(jnp.int32, (G, b.shape[0]), 0))
    oh = oh.astype(jnp.float32)
    acc_msg[...] += jnp.dot(oh, out_node, preferred_element_type=jnp.float32)
    acc_h[...] += jnp.dot(oh, h_ref[...], preferred_element_type=jnp.float32)
    acc_n[0, :] += jnp.sum(oh, axis=1)

    @pl.when(i == pl.num_programs(0) - 1)
    def _():
        nb = acc_n[0, :]
        pooled = (acc_msg[...]
                  + jnp.dot(acc_h[...], root_w_ref[...],
                            preferred_element_type=jnp.float32)
                  + nb[:, None] * root_b_ref[...])
        pooled = pooled / jnp.maximum(nb, 1.0)[:, None]
        z = jnp.maximum(
            jnp.dot(pooled, lin1_w_ref[...],
                    preferred_element_type=jnp.float32) + lin1_b_ref[...],
            0.0)
        logits = jnp.dot(z, lin2_w_ref[...],
                         preferred_element_type=jnp.float32) + lin2_b_ref[...]
        m = jnp.max(logits, axis=1, keepdims=True)
        s = jnp.log(jnp.sum(jnp.exp(logits - m), axis=1, keepdims=True))
        o_ref[...] = logits - m - s


def _head(parts, h, batch3, root_w, root_b, lin1_w, lin1_b, lin2_w, lin2_b):
    nb = 10
    rows = N // nb
    return pl.pallas_call(
        _head_body,
        grid=(nb,),
        in_specs=[
            pl.BlockSpec((2, rows, W), lambda i: (0, i, 0)),
            pl.BlockSpec((rows, D_IN), lambda i: (i, 0)),
            pl.BlockSpec((1, 1, rows), lambda i: (i, 0, 0)),
            pl.BlockSpec((D_IN, H), lambda i: (0, 0)),
            pl.BlockSpec((1, H), lambda i: (0, 0)),
            pl.BlockSpec((H, H), lambda i: (0, 0)),
            pl.BlockSpec((1, H), lambda i: (0, 0)),
            pl.BlockSpec((H, C), lambda i: (0, 0)),
            pl.BlockSpec((1, C), lambda i: (0, 0)),
        ],
        out_specs=pl.BlockSpec((G, C), lambda i: (0, 0)),
        out_shape=jax.ShapeDtypeStruct((G, C), jnp.float32),
        scratch_shapes=[
            pltpu.VMEM((G, H), jnp.float32),
            pltpu.VMEM((G, D_IN), jnp.float32),
            pltpu.VMEM((1, G), jnp.float32),
        ],
    )(parts, h, batch3, root_w, root_b, lin1_w, lin1_b, lin2_w, lin2_b)


def kernel(x, pos, edge_index, edge_attr, batch, g, mu, sigma, root_w, root_b,
           lin1_w, lin1_b, lin2_w, lin2_b):
    h = jnp.concatenate([pos, x], axis=1)
    hg = _compute_hg(h, g)
    hgp = jax.lax.bitcast_convert_type(
        hg.reshape(N, K * H // 2, 2), jnp.int32)

    src = edge_index[0].astype(jnp.int32)
    dst = edge_index[1].astype(jnp.int32)
    ea_flat = edge_attr.reshape(-1)

    inv0 = -0.5 / (1e-15 + sigma[:, 0] ** 2)
    inv1 = -0.5 / (1e-15 + sigma[:, 1] ** 2)
    prm = jnp.concatenate([mu[:, 0], mu[:, 1], inv0, inv1])
    prm = jnp.broadcast_to(prm[:, None], (4 * K, 16)).astype(jnp.float32)

    parts = _edge_aggregate(hgp, src, dst, ea_flat, prm)

    batch3 = batch.astype(jnp.int32).reshape(10, 1, N // 10)
    return _head(parts, h, batch3, root_w,
                 root_b.reshape(1, H), lin1_w, lin1_b.reshape(1, H),
                 lin2_w, lin2_b.reshape(1, C))
